# baseline (speedup 1.0000x reference)
import functools

import jax
import jax.numpy as jnp
from jax import lax
from jax.experimental import pallas as pl
from jax.experimental.pallas import tpu as pltpu
from jax.experimental.pallas import tpu_sc as plsc

INPUT_DIM = 4096
NUM_EXPERTS = 64
K = 8
TOKEN_BLOCK = 1024

N_TOK = 32768
NC = 2
NS = 16
NW = NC * NS
L = 16
TPW = N_TOK // NW
N_CHUNK = TPW // L


def _matmul_kernel(x_ref, w_ref, b_ref, o_ref):
    x = x_ref[...]
    w = w_ref[...]
    o_ref[...] = jax.lax.dot_general(
        w, x, (((1,), (1,)), ((), ())),
        preferred_element_type=jnp.float32) + b_ref[...]


def _tc_logits(x, W, b):
    n_tokens = x.shape[0]
    grid = (n_tokens // TOKEN_BLOCK,)
    return pl.pallas_call(
        _matmul_kernel,
        grid=grid,
        in_specs=[
            pl.BlockSpec((TOKEN_BLOCK, INPUT_DIM), lambda i: (i, 0)),
            pl.BlockSpec((NUM_EXPERTS, INPUT_DIM), lambda i: (0, 0)),
            pl.BlockSpec((NUM_EXPERTS, 1), lambda i: (0, 0)),
        ],
        out_specs=pl.BlockSpec((NUM_EXPERTS, TOKEN_BLOCK), lambda i: (0, i)),
        out_shape=jax.ShapeDtypeStruct((NUM_EXPERTS, n_tokens), jnp.float32),
        compiler_params=pltpu.CompilerParams(
            dimension_semantics=("parallel",)),
    )(x, W, b.reshape(NUM_EXPERTS, 1))


def _argmax_tree(vals, idxs):
    while len(vals) > 1:
        nv, ni = [], []
        for a in range(0, len(vals) - 1, 2):
            cond = vals[a] >= vals[a + 1]
            nv.append(jnp.where(cond, vals[a], vals[a + 1]))
            ni.append(jnp.where(cond, idxs[a], idxs[a + 1]))
        if len(vals) % 2:
            nv.append(vals[-1])
            ni.append(idxs[-1])
        vals, idxs = nv, ni
    return vals[0], idxs[0]


SUB = 256
N_SUB = TPW // SUB


def _sc_tail_body(logits_hbm, out_hbm, in_v, w_v, out_v):
    wid = lax.axis_index("s") * NC + lax.axis_index("c")
    zeros = jnp.zeros((L,), jnp.float32)

    def chunk(c, _):
        off = c * L
        m = in_v[0, pl.ds(off, L)]
        for e in range(1, NUM_EXPERTS):
            m = jnp.maximum(m, in_v[e, pl.ds(off, L)])
        s = zeros
        for e in range(NUM_EXPERTS):
            ev = jnp.exp(in_v[e, pl.ds(off, L)] - m)
            w_v[pl.ds(e * L, L)] = ev
            s = s + ev
        rcp = 1.0 / s
        picked = []
        for _ in range(K):
            vals = [w_v[pl.ds(e * L, L)] for e in range(NUM_EXPERTS)]
            idxs = [jnp.full((L,), e, jnp.int32) for e in range(NUM_EXPERTS)]
            mval, midx = _argmax_tree(vals, idxs)
            picked.append((mval, midx))
            for e in range(NUM_EXPERTS):
                w_v[pl.ds(e * L, L)] = jnp.where(midx == e, 0.0, vals[e])
        for e in range(NUM_EXPERTS):
            acc = zeros
            for mval, midx in picked:
                acc = acc + jnp.where(midx == e, mval, zeros)
            out_v[e, pl.ds(off, L)] = acc * rcp
        return ()

    def sub_block(sb, _):
        base = wid * TPW + sb * SUB
        pltpu.sync_copy(logits_hbm.at[:, pl.ds(base, SUB)], in_v)
        lax.fori_loop(0, SUB // L, chunk, ())
        pltpu.sync_copy(out_v, out_hbm.at[:, pl.ds(base, SUB)])
        return ()

    lax.fori_loop(0, N_SUB, sub_block, ())


def _sc_tail(logits_t):
    mesh = plsc.VectorSubcoreMesh(core_axis_name="c", subcore_axis_name="s")
    fn = functools.partial(
        pl.kernel, mesh=mesh,
        out_type=jax.ShapeDtypeStruct((NUM_EXPERTS, N_TOK), jnp.float32),
        scratch_types=[
            pltpu.VMEM((NUM_EXPERTS, SUB), jnp.float32),
            pltpu.VMEM((NUM_EXPERTS * L,), jnp.float32),
            pltpu.VMEM((NUM_EXPERTS, SUB), jnp.float32),
        ],
    )(_sc_tail_body)
    return fn(logits_t)


@jax.jit
def kernel(x, W, b):
    logits_t = _tc_logits(x, W, b)
    return _sc_tail(logits_t).T


# speedup vs baseline: 2.7614x; 2.7614x over previous
import jax
import jax.numpy as jnp
from jax.experimental import pallas as pl
from jax.experimental.pallas import tpu as pltpu

INPUT_DIM = 4096
NUM_EXPERTS = 64
K = 8
TOKEN_BLOCK = 2048
D_CHUNK = 2048
N_D = INPUT_DIM // D_CHUNK


def _router_kernel(x_ref, w_ref, b_ref, o_ref, acc_ref):
    j = pl.program_id(1)
    part = jax.lax.dot_general(
        w_ref[...], x_ref[...], (((1,), (1,)), ((), ())),
        preferred_element_type=jnp.float32)

    @pl.when(j == 0)
    def _init():
        acc_ref[...] = part

    @pl.when(j > 0)
    def _accum():
        acc_ref[...] += part

    @pl.when(j == N_D - 1)
    def _tail():
        logits = acc_ref[...] + b_ref[...]

        m = jnp.max(logits, axis=0, keepdims=True)
        e = jnp.exp(logits - m)
        p = e / jnp.sum(e, axis=0, keepdims=True)

        bt = logits.shape[1]
        iota = jax.lax.broadcasted_iota(jnp.int32, (NUM_EXPERTS, bt), 0)
        v = logits
        sel = jnp.zeros_like(logits, dtype=jnp.bool_)
        for _ in range(K):
            rmax = jnp.max(v, axis=0, keepdims=True)
            first = jnp.min(jnp.where(v == rmax, iota, NUM_EXPERTS),
                            axis=0, keepdims=True)
            pick = iota == first
            sel = jnp.logical_or(sel, pick)
            v = jnp.where(pick, -jnp.inf, v)

        o_ref[...] = jnp.where(sel, p, 0.0).T


@jax.jit
def kernel(x, W, b):
    n_tokens = x.shape[0]
    grid = (n_tokens // TOKEN_BLOCK, N_D)
    return pl.pallas_call(
        _router_kernel,
        grid=grid,
        in_specs=[
            pl.BlockSpec((TOKEN_BLOCK, D_CHUNK), lambda i, j: (i, j)),
            pl.BlockSpec((NUM_EXPERTS, D_CHUNK), lambda i, j: (0, j)),
            pl.BlockSpec((NUM_EXPERTS, 1), lambda i, j: (0, 0)),
        ],
        out_specs=pl.BlockSpec((TOKEN_BLOCK, NUM_EXPERTS),
                               lambda i, j: (i, 0)),
        out_shape=jax.ShapeDtypeStruct((n_tokens, NUM_EXPERTS), jnp.float32),
        scratch_shapes=[pltpu.VMEM((NUM_EXPERTS, TOKEN_BLOCK), jnp.float32)],
        compiler_params=pltpu.CompilerParams(
            dimension_semantics=("parallel", "arbitrary")),
    )(x, W, b.reshape(NUM_EXPERTS, 1))


# speedup vs baseline: 2.9858x; 1.0813x over previous
import jax
import jax.numpy as jnp
from jax.experimental import pallas as pl
from jax.experimental.pallas import tpu as pltpu

INPUT_DIM = 4096
NUM_EXPERTS = 64
K = 8
HALF_BLOCK = 512
TOKEN_BLOCK = 2 * HALF_BLOCK


def _tail(logits):
    m = jnp.max(logits, axis=0, keepdims=True)
    e = jnp.exp(logits - m)
    p = e / jnp.sum(e, axis=0, keepdims=True)

    bt = logits.shape[1]
    iota = jax.lax.broadcasted_iota(jnp.int32, (NUM_EXPERTS, bt), 0)
    v = logits
    sel = jnp.zeros_like(logits, dtype=jnp.bool_)
    for _ in range(K):
        rmax = jnp.max(v, axis=0, keepdims=True)
        first = jnp.min(jnp.where(v == rmax, iota, NUM_EXPERTS),
                        axis=0, keepdims=True)
        pick = iota == first
        sel = jnp.logical_or(sel, pick)
        v = jnp.where(pick, -jnp.inf, v)

    return jnp.where(sel, p, 0.0).T


def _router_kernel(x0_ref, x1_ref, w_ref, b_ref, o_ref):
    w = w_ref[...]
    b = b_ref[...]
    dims = (((1,), (1,)), ((), ()))
    logits0 = jax.lax.dot_general(
        w, x0_ref[...], dims, preferred_element_type=jnp.float32) + b
    o_ref[:HALF_BLOCK, :] = _tail(logits0)
    logits1 = jax.lax.dot_general(
        w, x1_ref[...], dims, preferred_element_type=jnp.float32) + b
    o_ref[HALF_BLOCK:, :] = _tail(logits1)


@jax.jit
def kernel(x, W, b):
    n_tokens = x.shape[0]
    grid = (n_tokens // TOKEN_BLOCK,)
    return pl.pallas_call(
        _router_kernel,
        grid=grid,
        in_specs=[
            pl.BlockSpec((HALF_BLOCK, INPUT_DIM), lambda i: (2 * i, 0)),
            pl.BlockSpec((HALF_BLOCK, INPUT_DIM), lambda i: (2 * i + 1, 0)),
            pl.BlockSpec((NUM_EXPERTS, INPUT_DIM), lambda i: (0, 0)),
            pl.BlockSpec((NUM_EXPERTS, 1), lambda i: (0, 0)),
        ],
        out_specs=pl.BlockSpec((TOKEN_BLOCK, NUM_EXPERTS), lambda i: (i, 0)),
        out_shape=jax.ShapeDtypeStruct((n_tokens, NUM_EXPERTS), jnp.float32),
        compiler_params=pltpu.CompilerParams(
            dimension_semantics=("parallel",)),
    )(x, x, W, b.reshape(NUM_EXPERTS, 1))
